# Initial kernel scaffold; baseline (speedup 1.0000x reference)
import jax
import jax.numpy as jnp
from jax.experimental import pallas as pl


def kernel(pts, plane_yx, plane_zx, plane_zy, W_dec, b_dec):
    raise NotImplementedError("write your pallas kernel here")


# speedup vs baseline: 13.0236x; 13.0236x over previous
import functools

import jax
import jax.numpy as jnp
from jax import lax
from jax.experimental import pallas as pl
from jax.experimental.pallas import tpu as pltpu
from jax.experimental.pallas import tpu_sc as plsc

NC = 2
NS = 16
L = 16
NW = NC * NS

C = 8
BSUB = 128
G = BSUB // L
CB = 8192


def _padded_cells(H, W):
    HpW = (H + 2) * W
    HpW += HpW % 2
    return HpW


def _make_tc_transpose(HW, HpW):
    grid = (HpW + CB - 1) // CB

    def body(in_ref, out_ref):
        j = pl.program_id(0)
        t = in_ref[...]
        ir = lax.broadcasted_iota(jnp.int32, (C, C), 0)
        ic = lax.broadcasted_iota(jnp.int32, (C, C), 1)
        eye = jnp.where(ir == ic, 1.0, 0.0).astype(jnp.float32)
        tt = lax.dot_general(t, eye, (((0,), (0,)), ((), ())),
                             precision=lax.Precision.HIGHEST,
                             preferred_element_type=jnp.float32)
        cid = lax.broadcasted_iota(jnp.int32, (CB, C), 0) + j * CB
        out_ref[...] = jnp.where(cid < HW, tt, 0.0)

    jmax = (HW - 1) // CB
    return pl.pallas_call(
        body,
        grid=(grid,),
        in_specs=[pl.BlockSpec((C, CB), lambda j: (0, jnp.minimum(j, jmax)))],
        out_specs=pl.BlockSpec((CB, C), lambda j: (j, 0)),
        out_shape=jax.ShapeDtypeStruct((HpW, C), jnp.float32),
    )


def _build_table(plane):
    c, H, W = plane.shape
    HW = H * W
    HpW = _padded_cells(H, W)
    tbl = _make_tc_transpose(HW, HpW)(plane.reshape(c, HW))
    return tbl.reshape(HpW // 2, 2 * C)


def _full(v):
    return jnp.full((L,), v, jnp.int32)


def _bilerp(rq0, rr0, rq1, rr1, pi, p8, p8b, m0, m1, cc, wx, wy):
    cv = _full(cc)
    cv8 = _full(cc + C)
    v00 = plsc.load_gather(rq0, [pi, p8 + cc])
    v10 = jnp.where(m0, plsc.load_gather(rq0, [pi, cv8]),
                    plsc.load_gather(rr0, [pi, cv]))
    v01 = plsc.load_gather(rq1, [pi, p8b + cc])
    v11 = jnp.where(m1, plsc.load_gather(rq1, [pi, cv8]),
                    plsc.load_gather(rr1, [pi, cv]))
    a = v00 + wx * (v10 - v00)
    b = v01 + wx * (v11 - v01)
    return a + wy * (b - a)


def _make_sc_kernel(N, Wyx, Hyx, Wzx, Hzx, Wzy, Hzy):
    BT = N // NW
    NSUB = BT // BSUB
    mesh = plsc.VectorSubcoreMesh(core_axis_name="c", subcore_axis_name="s")
    f32 = jnp.float32
    i32 = jnp.int32

    @functools.partial(
        pl.kernel,
        mesh=mesh,
        out_type=jax.ShapeDtypeStruct((N, 3), f32),
        compiler_params=pltpu.CompilerParams(
            needs_layout_passes=False, use_tc_tiling_on_sc=False),
        scratch_types=[
            pltpu.VMEM((BSUB,), f32),
            pltpu.VMEM((BSUB,), f32),
            pltpu.VMEM((BSUB,), f32),
            pltpu.VMEM((BSUB,), i32),
            pltpu.VMEM((BSUB,), i32),
            pltpu.VMEM((BSUB,), i32),
            pltpu.VMEM((BSUB,), i32),
            pltpu.VMEM((BSUB,), i32),
            pltpu.VMEM((BSUB,), i32),
            pltpu.VMEM((BSUB,), i32),
            pltpu.VMEM((BSUB,), i32),
            pltpu.VMEM((BSUB,), i32),
            pltpu.VMEM((BSUB,), i32),
            pltpu.VMEM((BSUB,), i32),
            pltpu.VMEM((BSUB,), i32),
            pltpu.VMEM((BSUB,), i32),
            pltpu.VMEM((BSUB,), i32),
            pltpu.VMEM((BSUB,), i32),
            pltpu.VMEM((BSUB,), f32),
            pltpu.VMEM((BSUB,), f32),
            pltpu.VMEM((BSUB,), f32),
            pltpu.VMEM((BSUB, 2 * C), f32),
            pltpu.VMEM((BSUB, 2 * C), f32),
            pltpu.VMEM((BSUB, 2 * C), f32),
            pltpu.VMEM((BSUB, 2 * C), f32),
            pltpu.VMEM((BSUB, 2 * C), f32),
            pltpu.VMEM((BSUB, 2 * C), f32),
            pltpu.VMEM((BSUB, 2 * C), f32),
            pltpu.VMEM((BSUB, 2 * C), f32),
            pltpu.VMEM((BSUB, 2 * C), f32),
            pltpu.VMEM((BSUB, 2 * C), f32),
            pltpu.VMEM((BSUB, 2 * C), f32),
            pltpu.VMEM((BSUB, 2 * C), f32),
            pltpu.VMEM((32,), f32),
            pltpu.VMEM((BSUB, 3), f32),
            pltpu.SemaphoreType.DMA,
        ],
    )
    def sc_kernel(xs_h, ys_h, zs_h, tyx_h, tzx_h, tzy_h, wb_h, out_h,
                  xs_v, ys_v, zs_v,
                  ia0q, ia0r, ia1q, ia1r, ib0q, ib0r, ib1q, ib1r,
                  ic0q, ic0r, ic1q, ic1r,
                  pa_v, pb_v, pc_v,
                  wx_v, wy_v, wz_v,
                  ra0q, ra0r, ra1q, ra1r, rb0q, rb0r, rb1q, rb1r,
                  rc0q, rc0r, rc1q, rc1r,
                  wb_v, out_v, sem):
        wid = lax.axis_index("s") * NC + lax.axis_index("c")
        pltpu.sync_copy(wb_h, wb_v)
        wsp = [[plsc.load_gather(wb_v, [_full(1 + cc * 3 + j)]) for j in range(3)]
               for cc in range(C)]
        bsp = [plsc.load_gather(wb_v, [_full(1 + C * 3 + j)]) for j in range(3)]

        def sub(s, carry):
            base = wid * BT + s * BSUB
            pltpu.sync_copy(xs_h.at[pl.ds(base, BSUB)], xs_v)
            pltpu.sync_copy(ys_h.at[pl.ds(base, BSUB)], ys_v)
            pltpu.sync_copy(zs_h.at[pl.ds(base, BSUB)], zs_v)
            for g in range(G):
                sl = pl.ds(g * L, L)
                x = xs_v[sl]
                y = ys_v[sl]
                z = zs_v[sl]
                ax = (x + 1.0) * 0.5 * (Wyx - 1)
                ay = (y + 1.0) * 0.5 * (Hyx - 1)
                az = (z + 1.0) * 0.5 * (Hzx - 1)
                xi = ax.astype(i32)
                yi = ay.astype(i32)
                zi = az.astype(i32)
                wx_v[sl] = ax - xi.astype(f32)
                wy_v[sl] = ay - yi.astype(f32)
                wz_v[sl] = az - zi.astype(f32)
                r0 = yi * Wyx + xi
                q = jnp.right_shift(r0, 1)
                ia0q[sl] = q
                ia0r[sl] = q + 1
                pa_v[sl] = jnp.bitwise_and(r0, 1) * 8
                q = jnp.right_shift(r0 + Wyx, 1)
                ia1q[sl] = q
                ia1r[sl] = q + 1
                r0 = zi * Wzx + xi
                q = jnp.right_shift(r0, 1)
                ib0q[sl] = q
                ib0r[sl] = q + 1
                pb_v[sl] = jnp.bitwise_and(r0, 1) * 8
                q = jnp.right_shift(r0 + Wzx, 1)
                ib1q[sl] = q
                ib1r[sl] = q + 1
                r0 = zi * Wzy + yi
                q = jnp.right_shift(r0, 1)
                ic0q[sl] = q
                ic0r[sl] = q + 1
                pc_v[sl] = jnp.bitwise_and(r0, 1) * 8
                q = jnp.right_shift(r0 + Wzy, 1)
                ic1q[sl] = q
                ic1r[sl] = q + 1
            cps = [
                pltpu.async_copy(tyx_h.at[ia0q], ra0q, sem),
                pltpu.async_copy(tyx_h.at[ia0r], ra0r, sem),
                pltpu.async_copy(tyx_h.at[ia1q], ra1q, sem),
                pltpu.async_copy(tyx_h.at[ia1r], ra1r, sem),
                pltpu.async_copy(tzx_h.at[ib0q], rb0q, sem),
                pltpu.async_copy(tzx_h.at[ib0r], rb0r, sem),
                pltpu.async_copy(tzx_h.at[ib1q], rb1q, sem),
                pltpu.async_copy(tzx_h.at[ib1r], rb1r, sem),
                pltpu.async_copy(tzy_h.at[ic0q], rc0q, sem),
                pltpu.async_copy(tzy_h.at[ic0r], rc0r, sem),
                pltpu.async_copy(tzy_h.at[ic1q], rc1q, sem),
                pltpu.async_copy(tzy_h.at[ic1r], rc1r, sem),
            ]
            for cp in cps:
                cp.wait()
            for g in range(G):
                sl = pl.ds(g * L, L)
                pi = lax.iota(i32, L) + g * L
                wx = wx_v[sl]
                wy = wy_v[sl]
                wz = wz_v[sl]
                p8a = pa_v[sl]
                p8b = pb_v[sl]
                p8c = pc_v[sl]
                p8a1 = (8 - p8a) if Wyx % 2 else p8a
                p8b1 = (8 - p8b) if Wzx % 2 else p8b
                p8c1 = (8 - p8c) if Wzy % 2 else p8c
                ma0 = p8a == 0
                ma1 = p8a1 == 0
                mb0 = p8b == 0
                mb1 = p8b1 == 0
                mc0 = p8c == 0
                mc1 = p8c1 == 0
                acc = [bsp[0], bsp[1], bsp[2]]
                for cc in range(C):
                    fa = _bilerp(ra0q, ra0r, ra1q, ra1r, pi, p8a, p8a1,
                                 ma0, ma1, cc, wx, wy)
                    fb = _bilerp(rb0q, rb0r, rb1q, rb1r, pi, p8b, p8b1,
                                 mb0, mb1, cc, wx, wz)
                    fc = _bilerp(rc0q, rc0r, rc1q, rc1r, pi, p8c, p8c1,
                                 mc0, mc1, cc, wy, wz)
                    f = fa * fb * fc
                    for j in range(3):
                        acc[j] = acc[j] + f * wsp[cc][j]
                for j in range(3):
                    plsc.store_scatter(out_v, [pi, _full(j)], acc[j])
            pltpu.sync_copy(out_v, out_h.at[pl.ds(base, BSUB)])
            return carry

        lax.fori_loop(0, NSUB, sub, 0)

    return sc_kernel


def kernel(pts, plane_yx, plane_zx, plane_zy, W_dec, b_dec):
    N = pts.shape[0]
    _, Hyx, Wyx = plane_yx.shape
    _, Hzx, Wzx = plane_zx.shape
    _, Hzy, Wzy = plane_zy.shape
    tyx = _build_table(plane_yx)
    tzx = _build_table(plane_zx)
    tzy = _build_table(plane_zy)
    xs = pts[:, 0]
    ys = pts[:, 1]
    zs = pts[:, 2]
    wb = jnp.concatenate(
        [jnp.zeros((1,), jnp.float32), W_dec.reshape(-1), b_dec,
         jnp.zeros((4,), jnp.float32)], axis=0)
    sc = _make_sc_kernel(N, Wyx, Hyx, Wzx, Hzx, Wzy, Hzy)
    return sc(xs, ys, zs, tyx, tzx, tzy, wb)
